# Initial kernel scaffold; baseline (speedup 1.0000x reference)
import jax
import jax.numpy as jnp
from jax.experimental import pallas as pl


def kernel(x_H, x_C, x_Others, ei_H_H, ei_H_C, ei_H_Others, ei_C_H, ei_C_C, ei_C_Others, ei_Others_H, ei_Others_C, ei_Others_Others, enc1_W_H, enc1_b_H, enc2_W_H, enc2_b_H, enc1_W_C, enc1_b_C, enc2_W_C, enc2_b_C, enc1_W_Others, enc1_b_Others, enc2_W_Others, enc2_b_Others, rel_W, rel_b, root_W, pred_W_H, pred_b_H, pred_W_C, pred_b_C):
    raise NotImplementedError("write your pallas kernel here")


# speedup vs baseline: 6.8620x; 6.8620x over previous
import functools

import jax
import jax.numpy as jnp
from jax import lax
from jax.experimental import pallas as pl
from jax.experimental.pallas import tpu as pltpu
from jax.experimental.pallas import tpu_sc as plsc

_NT = ("H", "C", "Others")
_N = {"H": 50000, "C": 30000, "Others": 20000}
_E = 400000
_PE = 401408
_G = 98
_BLK = 2000

_ACC = {"H": 50176, "C": 30208, "Others": 20096}


def _encoder(x, w1, b1, w2, b2):
    n = x.shape[0]

    def body(x_ref, w1_ref, b1_ref, w2_ref, b2_ref, o_ref):
        z = lax.dot_general(x_ref[...], w1_ref[...],
                            (((1,), (1,)), ((), ())),
                            preferred_element_type=jnp.float32)
        z = jnp.maximum(z + b1_ref[...], 0.0)
        z = lax.dot_general(z, w2_ref[...], (((1,), (1,)), ((), ())),
                            preferred_element_type=jnp.float32)
        o_ref[...] = jnp.maximum(z + b2_ref[...], 0.0)

    return pl.pallas_call(
        body,
        grid=(n // _BLK,),
        in_specs=[
            pl.BlockSpec((_BLK, 128), lambda i: (i, 0)),
            pl.BlockSpec((32, 128), lambda i: (0, 0)),
            pl.BlockSpec((1, 32), lambda i: (0, 0)),
            pl.BlockSpec((16, 32), lambda i: (0, 0)),
            pl.BlockSpec((1, 16), lambda i: (0, 0)),
        ],
        out_specs=pl.BlockSpec((_BLK, 16), lambda i: (i, 0)),
        out_shape=jax.ShapeDtypeStruct((n, 16), jnp.float32),
    )(x, w1, b1.reshape(1, 32), w2, b2.reshape(1, 16))


def _transform(h, wstack, bstack):
    n = h.shape[0]
    k = wstack.shape[0]

    def body(h_ref, w_ref, b_ref, o_ref):
        z = lax.dot_general(h_ref[...], w_ref[0], (((1,), (1,)), ((), ())),
                            preferred_element_type=jnp.float32)
        o_ref[0] = z + b_ref[0]

    return pl.pallas_call(
        body,
        grid=(k, n // _BLK),
        in_specs=[
            pl.BlockSpec((_BLK, 16), lambda j, i: (i, 0)),
            pl.BlockSpec((1, 16, 16), lambda j, i: (j, 0, 0)),
            pl.BlockSpec((1, 1, 16), lambda j, i: (j, 0, 0)),
        ],
        out_specs=pl.BlockSpec((1, _BLK, 16), lambda j, i: (j, i, 0)),
        out_shape=jax.ShapeDtypeStruct((k, n, 16), jnp.float32),
    )(h, wstack, bstack.reshape(k, 1, 16))


def _combine(partials, base):
    n = base.shape[0]

    def body(p_ref, b_ref, o_ref):
        o_ref[...] = jnp.maximum(p_ref[0] + p_ref[1] + b_ref[...], 0.0)

    return pl.pallas_call(
        body,
        grid=(n // _BLK,),
        in_specs=[
            pl.BlockSpec((2, _BLK, 16), lambda i: (0, i, 0)),
            pl.BlockSpec((_BLK, 16), lambda i: (i, 0)),
        ],
        out_specs=pl.BlockSpec((_BLK, 16), lambda i: (i, 0)),
        out_shape=jax.ShapeDtypeStruct((n, 16), jnp.float32),
    )(partials, base)


def _final(partials, base, pw, pb):
    n = base.shape[0]

    def body(p_ref, b_ref, w_ref, pb_ref, o_ref):
        h = jnp.maximum(p_ref[0] + p_ref[1] + b_ref[...], 0.0)
        z = jnp.sum(h * w_ref[...], axis=1, keepdims=True)
        o_ref[...] = z + pb_ref[0, 0]

    return pl.pallas_call(
        body,
        grid=(n // _BLK,),
        in_specs=[
            pl.BlockSpec((2, _BLK, 16), lambda i: (0, i, 0)),
            pl.BlockSpec((_BLK, 16), lambda i: (i, 0)),
            pl.BlockSpec((1, 16), lambda i: (0, 0)),
            pl.BlockSpec((1, 1), lambda i: (0, 0)),
        ],
        out_specs=pl.BlockSpec((_BLK, 1), lambda i: (i, 0)),
        out_shape=jax.ShapeDtypeStruct((n, 1), jnp.float32),
    )(partials, base, pw, pb.reshape(1, 1))


def _make_segsum(acc_rows):
    n_rel = 3
    rpt = acc_rows // 16
    zb = rpt // 4
    g32 = 32 * _G

    mesh = plsc.VectorSubcoreMesh(core_axis_name="c", subcore_axis_name="s")

    @functools.partial(
        pl.kernel, mesh=mesh,
        compiler_params=pltpu.CompilerParams(use_tc_tiling_on_sc=False),
        out_type=jax.ShapeDtypeStruct((2, acc_rows, 16), jnp.float32),
        scratch_types=[
            pltpu.VMEM((_G, 128), jnp.int32),
            pltpu.VMEM((_G, 128), jnp.int32),
            pltpu.VMEM((128, 16), jnp.float32),
            pltpu.VMEM((zb, 16), jnp.float32),
            pltpu.VMEM_SHARED((acc_rows, 16), jnp.float32),
            pltpu.SemaphoreType.DMA,
        ],
    )
    def k(t_h, t_c, t_o, srcg, dstg, out, src_v, dst_v, rows_v, buf_v, acc,
          sem):
        cid = lax.axis_index("c")
        sid = lax.axis_index("s")
        wid = cid * 16 + sid

        def zero_row(i, carry):
            buf_v[i] = jnp.zeros((16,), jnp.float32)
            return carry

        lax.fori_loop(0, zb, zero_row, 0)
        for j in range(4):
            pltpu.sync_copy(buf_v, acc.at[pl.ds(sid * rpt + j * zb, zb)])
        plsc.subcore_barrier()

        tables = (t_h, t_c, t_o)
        for r in range(n_rel):
            tbl = tables[r]
            base_row = r * g32 + wid * _G
            pltpu.sync_copy(srcg.at[pl.ds(base_row, _G)], src_v)
            pltpu.sync_copy(dstg.at[pl.ds(base_row, _G)], dst_v)

            def step(j, carry):
                pltpu.async_copy(tbl.at[src_v.at[j]], rows_v, sem).wait()
                pltpu.sync_copy(rows_v, acc.at[dst_v.at[j]], add=True)
                return carry

            lax.fori_loop(0, _G, step, 0)
        plsc.subcore_barrier()

        for j in range(4):
            row0 = sid * rpt + j * zb
            pltpu.sync_copy(acc.at[pl.ds(row0, zb)], buf_v)
            pltpu.sync_copy(buf_v, out.at[cid, pl.ds(row0, zb)])

    return k


@functools.cache
def _segsum_kernel(acc_rows):
    return _make_segsum(acc_rows)


def _segsum_call(d, t_h, t_c, t_o, srcg, dstg):
    return _segsum_kernel(_ACC[d])(t_h, t_c, t_o, srcg, dstg)


def _prep_indices(eis, d, slot):
    srcs, dsts = [], []
    pad_s = jnp.zeros((_PE - _E,), jnp.int32)
    pad_d = jnp.full((_PE - _E,), _N[d], jnp.int32)
    for s in _NT:
        ei = eis[(s, d)].astype(jnp.int32)
        srcs.append(jnp.concatenate([ei[0] + slot * _N[s], pad_s]))
        dsts.append(jnp.concatenate([ei[1], pad_d]))
    return (jnp.concatenate(srcs).reshape(-1, 128),
            jnp.concatenate(dsts).reshape(-1, 128))


def kernel(x_H, x_C, x_Others, ei_H_H, ei_H_C, ei_H_Others, ei_C_H, ei_C_C,
           ei_C_Others, ei_Others_H, ei_Others_C, ei_Others_Others,
           enc1_W_H, enc1_b_H, enc2_W_H, enc2_b_H,
           enc1_W_C, enc1_b_C, enc2_W_C, enc2_b_C,
           enc1_W_Others, enc1_b_Others, enc2_W_Others, enc2_b_Others,
           rel_W, rel_b, root_W, pred_W_H, pred_b_H, pred_W_C, pred_b_C):
    inp = dict(locals())
    xs = {t: inp[f"x_{t}"] for t in _NT}
    eis = {(s, d): inp[f"ei_{s}_{d}"] for s in _NT for d in _NT}

    h = {t: _encoder(xs[t], inp[f"enc1_W_{t}"], inp[f"enc1_b_{t}"],
                     inp[f"enc2_W_{t}"], inp[f"enc2_b_{t}"]) for t in _NT}

    idx1 = {d: _prep_indices(eis, d, {"H": 0, "C": 1, "Others": 2}[d])
            for d in _NT}
    idx2 = {d: _prep_indices(eis, d, {"H": 0, "C": 1}[d]) for d in ("H", "C")}

    tout = {}
    for ti, t in enumerate(_NT):
        root_sum = jnp.sum(root_W[0, ti::3], axis=0)
        bias = jnp.sum(rel_b[0, ti::3], axis=0)
        wst = jnp.concatenate([rel_W[0, 3 * ti:3 * ti + 3], root_sum[None]], 0)
        bst = jnp.concatenate([jnp.zeros((3, 16), jnp.float32), bias[None]], 0)
        tout[t] = _transform(h[t], wst, bst)

    tbls = [tout[t].reshape(-1, 16) for t in _NT]
    p1 = {d: _segsum_call(d, *tbls, *idx1[d]) for d in _NT}
    h1 = {t: _combine(p1[t], tout[t][3]) for t in _NT}

    tout2 = {}
    for ti, t in enumerate(_NT):
        mats = [rel_W[1, 3 * ti], rel_W[1, 3 * ti + 1]]
        if t != "Others":
            root_sum = jnp.sum(root_W[1, ti::3], axis=0)
            bias = jnp.sum(rel_b[1, ti::3], axis=0)
            wst = jnp.stack(mats + [root_sum])
            bst = jnp.concatenate(
                [jnp.zeros((2, 16), jnp.float32), bias[None]], 0)
        else:
            wst = jnp.stack(mats)
            bst = jnp.zeros((2, 16), jnp.float32)
        tout2[t] = _transform(h1[t], wst, bst)

    tbls2 = [tout2[t].reshape(-1, 16) for t in _NT]
    p2 = {d: _segsum_call(d, *tbls2, *idx2[d]) for d in ("H", "C")}

    out_H = _final(p2["H"], tout2["H"][2], pred_W_H, pred_b_H)
    out_C = _final(p2["C"], tout2["C"][2], pred_W_C, pred_b_C)
    return out_H, out_C


# speedup vs baseline: 8.7257x; 1.2716x over previous
import functools

import jax
import jax.numpy as jnp
from jax import lax
from jax.experimental import pallas as pl
from jax.experimental.pallas import tpu as pltpu
from jax.experimental.pallas import tpu_sc as plsc

_NT = ("H", "C", "Others")
_N = {"H": 50000, "C": 30000, "Others": 20000}
_E = 400000
_PE = 401408
_G = 98
_BLK = 2000

_ACC = {"H": 50176, "C": 30208, "Others": 20096}
_SEC = {"H": ((0,), (1,), (2,)), "C": ((0, 1), (2,)), "Others": ((0, 1, 2),)}


def _b16(x):
    return x.astype(jnp.bfloat16)


def _encoder(x, w1, b1, w2, b2):
    n = x.shape[0]

    def body(x_ref, w1_ref, b1_ref, w2_ref, b2_ref, o_ref):
        z = lax.dot_general(_b16(x_ref[...]), _b16(w1_ref[...]),
                            (((1,), (1,)), ((), ())),
                            preferred_element_type=jnp.float32)
        z = jnp.maximum(z + b1_ref[...], 0.0)
        z = lax.dot_general(_b16(z), _b16(w2_ref[...]),
                            (((1,), (1,)), ((), ())),
                            preferred_element_type=jnp.float32)
        o_ref[...] = jnp.maximum(z + b2_ref[...], 0.0)

    return pl.pallas_call(
        body,
        grid=(n // _BLK,),
        in_specs=[
            pl.BlockSpec((_BLK, 128), lambda i: (i, 0)),
            pl.BlockSpec((32, 128), lambda i: (0, 0)),
            pl.BlockSpec((1, 32), lambda i: (0, 0)),
            pl.BlockSpec((16, 32), lambda i: (0, 0)),
            pl.BlockSpec((1, 16), lambda i: (0, 0)),
        ],
        out_specs=pl.BlockSpec((_BLK, 16), lambda i: (i, 0)),
        out_shape=jax.ShapeDtypeStruct((n, 16), jnp.float32),
    )(x, w1, b1.reshape(1, 32), w2, b2.reshape(1, 16))


def _sumdot(z, w_ref, j):
    return lax.dot_general(_b16(z), _b16(w_ref[j]), (((1,), (1,)), ((), ())),
                           preferred_element_type=jnp.float32)


def _combine(parts, h, wstack, bias):
    n = h.shape[0]

    def body(p_ref, h_ref, w_ref, b_ref, o_ref):
        z = b_ref[...]
        for r in range(3):
            z = z + _sumdot(p_ref[r, 0] + p_ref[r, 1], w_ref, r)
            z = z + _sumdot(h_ref[...], w_ref, 3 + r)
        o_ref[...] = jnp.maximum(z, 0.0)

    return pl.pallas_call(
        body,
        grid=(n // _BLK,),
        in_specs=[
            pl.BlockSpec((3, 2, _BLK, 16), lambda i: (0, 0, i, 0)),
            pl.BlockSpec((_BLK, 16), lambda i: (i, 0)),
            pl.BlockSpec((6, 16, 16), lambda i: (0, 0, 0)),
            pl.BlockSpec((1, 16), lambda i: (0, 0)),
        ],
        out_specs=pl.BlockSpec((_BLK, 16), lambda i: (i, 0)),
        out_shape=jax.ShapeDtypeStruct((n, 16), jnp.float32),
    )(parts, h, wstack, bias.reshape(1, 16))


def _final(parts, h, wstack, bias, pw, pb):
    n = h.shape[0]

    def body(p_ref, h_ref, w_ref, b_ref, pw_ref, pb_ref, o_ref):
        z = b_ref[...]
        for r in range(3):
            z = z + _sumdot(p_ref[r, 0] + p_ref[r, 1], w_ref, r)
            z = z + _sumdot(h_ref[...], w_ref, 3 + r)
        h2 = jnp.maximum(z, 0.0)
        prod = _b16(h2).astype(jnp.float32) * _b16(pw_ref[...]).astype(jnp.float32)
        o_ref[...] = jnp.sum(prod, axis=1, keepdims=True) + pb_ref[0, 0]

    return pl.pallas_call(
        body,
        grid=(n // _BLK,),
        in_specs=[
            pl.BlockSpec((3, 2, _BLK, 16), lambda i: (0, 0, i, 0)),
            pl.BlockSpec((_BLK, 16), lambda i: (i, 0)),
            pl.BlockSpec((6, 16, 16), lambda i: (0, 0, 0)),
            pl.BlockSpec((1, 16), lambda i: (0, 0)),
            pl.BlockSpec((1, 16), lambda i: (0, 0)),
            pl.BlockSpec((1, 1), lambda i: (0, 0)),
        ],
        out_specs=pl.BlockSpec((_BLK, 1), lambda i: (i, 0)),
        out_shape=jax.ShapeDtypeStruct((n, 1), jnp.float32),
    )(parts, h, wstack, bias.reshape(1, 16), pw, pb.reshape(1, 1))


def _make_segsum(acc_rows, n_sec):
    tot_rows = n_sec * acc_rows
    rpt = tot_rows // 16
    nz = 4 if rpt % 4 == 0 else 2
    zb = rpt // nz
    g32 = 32 * _G

    mesh = plsc.VectorSubcoreMesh(core_axis_name="c", subcore_axis_name="s")

    pb = 7
    nb = _G // pb
    half_rows = pb * 128

    @functools.partial(
        pl.kernel, mesh=mesh,
        compiler_params=pltpu.CompilerParams(use_tc_tiling_on_sc=False),
        out_type=jax.ShapeDtypeStruct((2, tot_rows, 16), jnp.float32),
        scratch_types=[
            pltpu.VMEM((_G, 128), jnp.int32),
            pltpu.VMEM((_G, 128), jnp.int32),
            pltpu.VMEM((2, half_rows, 16), jnp.float32),
            pltpu.VMEM((zb, 16), jnp.float32),
            pltpu.VMEM_SHARED((tot_rows, 16), jnp.float32),
            pltpu.SemaphoreType.DMA,
            pltpu.SemaphoreType.DMA,
            pltpu.SemaphoreType.DMA,
            pltpu.SemaphoreType.DMA,
        ],
    )
    def k(*refs):
        tables = refs[:n_sec]
        srcg, dstg, out = refs[n_sec], refs[n_sec + 1], refs[n_sec + 2]
        src_v, dst_v, rows_v, buf_v, acc, sg0, sg1, ss0, ss1 = refs[n_sec + 3:]
        cid = lax.axis_index("c")
        sid = lax.axis_index("s")
        wid = cid * 16 + sid

        def zero_row(i, carry):
            buf_v[i] = jnp.zeros((16,), jnp.float32)
            return carry

        lax.fori_loop(0, zb, zero_row, 0)
        for j in range(nz):
            pltpu.sync_copy(buf_v, acc.at[pl.ds(sid * rpt + j * zb, zb)])
        plsc.subcore_barrier()

        for r in range(n_sec):
            tbl = tables[r]
            base_row = r * g32 + wid * _G
            pltpu.sync_copy(srcg.at[pl.ds(base_row, _G)], src_v)
            pltpu.sync_copy(dstg.at[pl.ds(base_row, _G)], dst_v)

            def gathers(b, p, sg):
                for t in range(pb):
                    pltpu.async_copy(
                        tbl.at[src_v.at[b * pb + t]],
                        rows_v.at[p, pl.ds(t * 128, 128)], sg)

            def scatters(b, p, ss):
                for t in range(pb):
                    pltpu.async_copy(
                        rows_v.at[p, pl.ds(t * 128, 128)],
                        acc.at[dst_v.at[b * pb + t]], ss, add=True)

            def drain_g(p, sg):
                pltpu.make_async_copy(
                    tbl.at[pl.ds(0, half_rows)], rows_v.at[p], sg).wait()

            def drain_s(p, ss):
                pltpu.make_async_copy(
                    rows_v.at[p], acc.at[pl.ds(0, half_rows)], ss).wait()

            gathers(0, 0, sg0)

            def pipe(i, carry):
                drain_g(0, sg0)
                scatters(2 * i, 0, ss0)

                @pl.when(i > 0)
                def _():
                    drain_s(1, ss1)
                gathers(2 * i + 1, 1, sg1)
                drain_g(1, sg1)
                scatters(2 * i + 1, 1, ss1)
                drain_s(0, ss0)

                @pl.when(i < nb // 2 - 1)
                def _():
                    gathers(2 * i + 2, 0, sg0)
                return carry

            lax.fori_loop(0, nb // 2, pipe, 0)
            drain_s(1, ss1)
        plsc.subcore_barrier()

        for j in range(nz):
            row0 = sid * rpt + j * zb
            pltpu.sync_copy(acc.at[pl.ds(row0, zb)], buf_v)
            pltpu.sync_copy(buf_v, out.at[cid, pl.ds(row0, zb)])

    return k


@functools.cache
def _segsum_kernel(acc_rows, n_sec):
    return _make_segsum(acc_rows, n_sec)


def _prep_indices(eis, d, srcs_in_call):
    acc = _ACC[d]
    srcs, dsts = [], []
    pad_s = jnp.zeros((_PE - _E,), jnp.int32)
    for sec, si in enumerate(srcs_in_call):
        s = _NT[si]
        ei = eis[(s, d)].astype(jnp.int32)
        srcs.append(jnp.concatenate([ei[0], pad_s]))
        dsts.append(jnp.concatenate(
            [ei[1] + sec * acc,
             jnp.full((_PE - _E,), sec * acc + _N[d], jnp.int32)]))
    return (jnp.concatenate(srcs).reshape(-1, 128),
            jnp.concatenate(dsts).reshape(-1, 128))


def _layer_partials(h, idx, dsts):
    out = {}
    for d in dsts:
        acc = _ACC[d]
        secs = []
        for srcs_in_call, (srcg, dstg) in zip(_SEC[d], idx[d]):
            tables = [h[_NT[si]] for si in srcs_in_call]
            p = _segsum_kernel(acc, len(srcs_in_call))(*tables, srcg, dstg)
            for sec in range(len(srcs_in_call)):
                secs.append(p[:, sec * acc:sec * acc + _N[d]])
        out[d] = jnp.stack(secs)
    return out


def _wstack(l, d, rel_W, root_W):
    di = _NT.index(d)
    rel_idx = [3 * si + di for si in range(3)]
    return jnp.stack([rel_W[l, r] for r in rel_idx]
                     + [root_W[l, r] for r in rel_idx])


def kernel(x_H, x_C, x_Others, ei_H_H, ei_H_C, ei_H_Others, ei_C_H, ei_C_C,
           ei_C_Others, ei_Others_H, ei_Others_C, ei_Others_Others,
           enc1_W_H, enc1_b_H, enc2_W_H, enc2_b_H,
           enc1_W_C, enc1_b_C, enc2_W_C, enc2_b_C,
           enc1_W_Others, enc1_b_Others, enc2_W_Others, enc2_b_Others,
           rel_W, rel_b, root_W, pred_W_H, pred_b_H, pred_W_C, pred_b_C):
    inp = dict(locals())
    xs = {t: inp[f"x_{t}"] for t in _NT}
    eis = {(s, d): inp[f"ei_{s}_{d}"] for s in _NT for d in _NT}

    h = {t: _encoder(xs[t], inp[f"enc1_W_{t}"], inp[f"enc1_b_{t}"],
                     inp[f"enc2_W_{t}"], inp[f"enc2_b_{t}"]) for t in _NT}

    idx = {d: [_prep_indices(eis, d, call) for call in _SEC[d]] for d in _NT}

    def bias(l, d):
        return jnp.sum(rel_b[l, _NT.index(d)::3], axis=0)

    p1 = _layer_partials(h, idx, _NT)
    h1 = {d: _combine(p1[d], h[d], _wstack(0, d, rel_W, root_W), bias(0, d))
          for d in _NT}

    p2 = _layer_partials(h1, idx, ("H", "C"))
    out_H = _final(p2["H"], h1["H"], _wstack(1, "H", rel_W, root_W),
                   bias(1, "H"), pred_W_H, pred_b_H)
    out_C = _final(p2["C"], h1["C"], _wstack(1, "C", rel_W, root_W),
                   bias(1, "C"), pred_W_C, pred_b_C)
    return out_H, out_C


# speedup vs baseline: 14.2278x; 1.6306x over previous
import functools

import jax
import jax.numpy as jnp
from jax import lax
from jax.experimental import pallas as pl
from jax.experimental.pallas import tpu as pltpu
from jax.experimental.pallas import tpu_sc as plsc

_NT = ("H", "C", "Others")
_N = {"H": 50000, "C": 30000, "Others": 20000}
_E = 400000
_PE = 401408
_G = 98
_BLK = 2000

_ACC = dict(_N)
_SEC = {"H": ((0,), (1,), (2,)), "C": ((0, 1), (2,)), "Others": ((0, 1, 2),)}


def _b16(x):
    return x.astype(jnp.bfloat16)


def _encoder(x, w1, b1, w2, b2):
    n = x.shape[0]

    nb = n // _BLK

    def body(x_ref, w1_ref, b1_ref, w2_ref, b2_ref, o_ref):
        z = lax.dot_general(_b16(x_ref[...]), _b16(w1_ref[...]),
                            (((1,), (1,)), ((), ())),
                            preferred_element_type=jnp.float32)
        z = jnp.maximum(z + b1_ref[...], 0.0)
        z = lax.dot_general(_b16(z), _b16(w2_ref[...]),
                            (((1,), (1,)), ((), ())),
                            preferred_element_type=jnp.float32)
        z = jnp.maximum(z + b2_ref[...], 0.0)
        o_ref[...] = jnp.where(pl.program_id(0) < nb, z, 0.0)

    return pl.pallas_call(
        body,
        grid=(nb + 1,),
        in_specs=[
            pl.BlockSpec((_BLK, 128), lambda i: (jnp.minimum(i, nb - 1), 0)),
            pl.BlockSpec((32, 128), lambda i: (0, 0)),
            pl.BlockSpec((1, 32), lambda i: (0, 0)),
            pl.BlockSpec((16, 32), lambda i: (0, 0)),
            pl.BlockSpec((1, 16), lambda i: (0, 0)),
        ],
        out_specs=pl.BlockSpec((_BLK, 16), lambda i: (i, 0)),
        out_shape=jax.ShapeDtypeStruct((n + _BLK, 16), jnp.float32),
    )(x, w1, b1.reshape(1, 32), w2, b2.reshape(1, 16))


def _sumdot(z, w_ref, j):
    return lax.dot_general(_b16(z), _b16(w_ref[j]), (((1,), (1,)), ((), ())),
                           preferred_element_type=jnp.float32)


def _part_specs(parts, nb):
    specs = [pl.BlockSpec(
        (2, _BLK, 16),
        functools.partial(lambda o, m, i: (0, jnp.minimum(i, m - 1) + o, 0),
                          off, nb))
             for _, off in parts]
    return [arr for arr, _ in parts], specs


def _combine(parts, h, wstack, bias):
    n = h.shape[0] - _BLK
    nb = n // _BLK
    arrs, pspecs = _part_specs(parts, nb)

    def body(pa_ref, pb_ref_, pc_ref, h_ref, w_ref, b_ref, o_ref):
        z = b_ref[...]
        for r, p_ref in enumerate((pa_ref, pb_ref_, pc_ref)):
            z = z + _sumdot(p_ref[0] + p_ref[1], w_ref, r)
            z = z + _sumdot(h_ref[...], w_ref, 3 + r)
        z = jnp.maximum(z, 0.0)
        o_ref[...] = jnp.where(pl.program_id(0) < nb, z, 0.0)

    return pl.pallas_call(
        body,
        grid=(nb + 1,),
        in_specs=pspecs + [
            pl.BlockSpec((_BLK, 16), lambda i: (jnp.minimum(i, nb - 1), 0)),
            pl.BlockSpec((6, 16, 16), lambda i: (0, 0, 0)),
            pl.BlockSpec((1, 16), lambda i: (0, 0)),
        ],
        out_specs=pl.BlockSpec((_BLK, 16), lambda i: (i, 0)),
        out_shape=jax.ShapeDtypeStruct((n + _BLK, 16), jnp.float32),
    )(*arrs, h, wstack, bias.reshape(1, 16))


def _final(parts, h, wstack, bias, pw, pb):
    n = h.shape[0] - _BLK
    arrs, pspecs = _part_specs(parts, n // _BLK)

    def body(pa_ref, pb_ref_, pc_ref, h_ref, w_ref, b_ref, pw_ref, pbias_ref,
             o_ref):
        z = b_ref[...]
        for r, p_ref in enumerate((pa_ref, pb_ref_, pc_ref)):
            z = z + _sumdot(p_ref[0] + p_ref[1], w_ref, r)
            z = z + _sumdot(h_ref[...], w_ref, 3 + r)
        h2 = jnp.maximum(z, 0.0)
        prod = _b16(h2).astype(jnp.float32) * _b16(pw_ref[...]).astype(jnp.float32)
        o_ref[...] = jnp.sum(prod, axis=1, keepdims=True) + pbias_ref[0, 0]

    return pl.pallas_call(
        body,
        grid=(n // _BLK,),
        in_specs=pspecs + [
            pl.BlockSpec((_BLK, 16), lambda i: (i, 0)),
            pl.BlockSpec((6, 16, 16), lambda i: (0, 0, 0)),
            pl.BlockSpec((1, 16), lambda i: (0, 0)),
            pl.BlockSpec((1, 16), lambda i: (0, 0)),
            pl.BlockSpec((1, 1), lambda i: (0, 0)),
        ],
        out_specs=pl.BlockSpec((_BLK, 1), lambda i: (i, 0)),
        out_shape=jax.ShapeDtypeStruct((n, 1), jnp.float32),
    )(*arrs, h, wstack, bias.reshape(1, 16), pw, pb.reshape(1, 1))


def _make_segsum(acc_rows, n_sec):
    tot_rows = n_sec * acc_rows
    rpt = tot_rows // 16
    zb = 625
    nz = rpt // zb
    assert nz * zb == rpt
    g32 = 32 * _G

    mesh = plsc.VectorSubcoreMesh(core_axis_name="c", subcore_axis_name="s")

    pb = 7
    nb = _G // pb
    half_rows = pb * 128

    @functools.partial(
        pl.kernel, mesh=mesh,
        compiler_params=pltpu.CompilerParams(use_tc_tiling_on_sc=False),
        out_type=jax.ShapeDtypeStruct((2, tot_rows, 16), jnp.float32),
        scratch_types=[
            pltpu.VMEM((_G, 128), jnp.int32),
            pltpu.VMEM((_G, 128), jnp.int32),
            pltpu.VMEM((2, half_rows, 16), jnp.float32),
            pltpu.VMEM((zb, 16), jnp.float32),
            pltpu.VMEM_SHARED((tot_rows, 16), jnp.float32),
            pltpu.SemaphoreType.DMA,
            pltpu.SemaphoreType.DMA,
            pltpu.SemaphoreType.DMA,
            pltpu.SemaphoreType.DMA,
        ],
    )
    def k(*refs):
        tables = refs[:n_sec]
        srcg, dstg, out = refs[n_sec], refs[n_sec + 1], refs[n_sec + 2]
        src_v, dst_v, rows_v, buf_v, acc, sg0, sg1, ss0, ss1 = refs[n_sec + 3:]
        cid = lax.axis_index("c")
        sid = lax.axis_index("s")
        wid = cid * 16 + sid

        def zero_row(i, carry):
            buf_v[i] = jnp.zeros((16,), jnp.float32)
            return carry

        lax.fori_loop(0, zb, zero_row, 0)
        for j in range(nz):
            pltpu.sync_copy(buf_v, acc.at[pl.ds(sid * rpt + j * zb, zb)])
        plsc.subcore_barrier()

        for r in range(n_sec):
            tbl = tables[r]
            base_row = r * g32 + wid * _G
            pltpu.sync_copy(srcg.at[pl.ds(base_row, _G)], src_v)
            pltpu.sync_copy(dstg.at[pl.ds(base_row, _G)], dst_v)

            def gathers(b, p, sg):
                for t in range(pb):
                    pltpu.async_copy(
                        tbl.at[src_v.at[b * pb + t]],
                        rows_v.at[p, pl.ds(t * 128, 128)], sg)

            def scatters(b, p, ss):
                for t in range(pb):
                    pltpu.async_copy(
                        rows_v.at[p, pl.ds(t * 128, 128)],
                        acc.at[dst_v.at[b * pb + t]], ss, add=True)

            def drain_g(p, sg):
                pltpu.make_async_copy(
                    tbl.at[pl.ds(0, half_rows)], rows_v.at[p], sg).wait()

            def drain_s(p, ss):
                pltpu.make_async_copy(
                    rows_v.at[p], acc.at[pl.ds(0, half_rows)], ss).wait()

            gathers(0, 0, sg0)

            def pipe(i, carry):
                drain_g(0, sg0)
                scatters(2 * i, 0, ss0)

                @pl.when(i > 0)
                def _():
                    drain_s(1, ss1)
                gathers(2 * i + 1, 1, sg1)
                drain_g(1, sg1)
                scatters(2 * i + 1, 1, ss1)
                drain_s(0, ss0)

                @pl.when(i < nb // 2 - 1)
                def _():
                    gathers(2 * i + 2, 0, sg0)
                return carry

            lax.fori_loop(0, nb // 2, pipe, 0)
            drain_s(1, ss1)
        plsc.subcore_barrier()

        for j in range(nz):
            row0 = sid * rpt + j * zb
            pltpu.sync_copy(acc.at[pl.ds(row0, zb)], buf_v)
            pltpu.sync_copy(buf_v, out.at[cid, pl.ds(row0, zb)])

    return k


@functools.cache
def _segsum_kernel(acc_rows, n_sec):
    return _make_segsum(acc_rows, n_sec)


def _prep_indices(eis, d, srcs_in_call):
    acc = _ACC[d]
    srcs, dsts = [], []
    for sec, si in enumerate(srcs_in_call):
        s = _NT[si]
        ei = eis[(s, d)].astype(jnp.int32)
        srcs.append(jnp.concatenate(
            [ei[0], jnp.full((_PE - _E,), _N[s], jnp.int32)]))
        dsts.append(jnp.concatenate(
            [ei[1] + sec * acc,
             jnp.full((_PE - _E,), sec * acc, jnp.int32)]))
    return (jnp.concatenate(srcs).reshape(-1, 128),
            jnp.concatenate(dsts).reshape(-1, 128))


def _layer_partials(h, idx, dsts):
    out = {}
    for d in dsts:
        acc = _ACC[d]
        secs = []
        for srcs_in_call, (srcg, dstg) in zip(_SEC[d], idx[d]):
            tables = [h[_NT[si]] for si in srcs_in_call]
            p = _segsum_kernel(acc, len(srcs_in_call))(*tables, srcg, dstg)
            for sec in range(len(srcs_in_call)):
                secs.append((p, sec * acc // _BLK))
        out[d] = secs
    return out


def _wstack(l, d, rel_W, root_W):
    di = _NT.index(d)
    rel_idx = [3 * si + di for si in range(3)]
    return jnp.stack([rel_W[l, r] for r in rel_idx]
                     + [root_W[l, r] for r in rel_idx])


def kernel(x_H, x_C, x_Others, ei_H_H, ei_H_C, ei_H_Others, ei_C_H, ei_C_C,
           ei_C_Others, ei_Others_H, ei_Others_C, ei_Others_Others,
           enc1_W_H, enc1_b_H, enc2_W_H, enc2_b_H,
           enc1_W_C, enc1_b_C, enc2_W_C, enc2_b_C,
           enc1_W_Others, enc1_b_Others, enc2_W_Others, enc2_b_Others,
           rel_W, rel_b, root_W, pred_W_H, pred_b_H, pred_W_C, pred_b_C):
    inp = dict(locals())
    xs = {t: inp[f"x_{t}"] for t in _NT}
    eis = {(s, d): inp[f"ei_{s}_{d}"] for s in _NT for d in _NT}

    h = {t: _encoder(xs[t], inp[f"enc1_W_{t}"], inp[f"enc1_b_{t}"],
                     inp[f"enc2_W_{t}"], inp[f"enc2_b_{t}"]) for t in _NT}

    idx = {d: [_prep_indices(eis, d, call) for call in _SEC[d]] for d in _NT}

    def bias(l, d):
        return jnp.sum(rel_b[l, _NT.index(d)::3], axis=0)

    p1 = _layer_partials(h, idx, _NT)
    h1 = {d: _combine(p1[d], h[d], _wstack(0, d, rel_W, root_W), bias(0, d))
          for d in _NT}

    p2 = _layer_partials(h1, idx, ("H", "C"))
    out_H = _final(p2["H"], h1["H"], _wstack(1, "H", rel_W, root_W),
                   bias(1, "H"), pred_W_H, pred_b_H)
    out_C = _final(p2["C"], h1["C"], _wstack(1, "C", rel_W, root_W),
                   bias(1, "C"), pred_W_C, pred_b_C)
    return out_H, out_C


# speedup vs baseline: 14.2563x; 1.0020x over previous
import functools

import jax
import jax.numpy as jnp
from jax import lax
from jax.experimental import pallas as pl
from jax.experimental.pallas import tpu as pltpu
from jax.experimental.pallas import tpu_sc as plsc

_NT = ("H", "C", "Others")
_N = {"H": 50000, "C": 30000, "Others": 20000}
_E = 400000
_PE = 401408
_G = 98
_BLK = 2000

_ACC = dict(_N)
_SEC = {"H": ((0,), (1,), (2,)), "C": ((0, 1), (2,)), "Others": ((0, 1, 2),)}


def _b16(x):
    return x.astype(jnp.bfloat16)


def _encoder(x, w1, b1, w2, b2):
    n = x.shape[0]

    nb = n // _BLK

    def body(x_ref, w1_ref, b1_ref, w2_ref, b2_ref, o_ref):
        z = lax.dot_general(_b16(x_ref[...]), _b16(w1_ref[...]),
                            (((1,), (1,)), ((), ())),
                            preferred_element_type=jnp.float32)
        z = jnp.maximum(z + b1_ref[...], 0.0)
        z = lax.dot_general(_b16(z), _b16(w2_ref[...]),
                            (((1,), (1,)), ((), ())),
                            preferred_element_type=jnp.float32)
        z = jnp.maximum(z + b2_ref[...], 0.0)
        o_ref[...] = jnp.where(pl.program_id(0) < nb, z, 0.0)

    return pl.pallas_call(
        body,
        grid=(nb + 1,),
        in_specs=[
            pl.BlockSpec((_BLK, 128), lambda i: (jnp.minimum(i, nb - 1), 0)),
            pl.BlockSpec((32, 128), lambda i: (0, 0)),
            pl.BlockSpec((1, 32), lambda i: (0, 0)),
            pl.BlockSpec((16, 32), lambda i: (0, 0)),
            pl.BlockSpec((1, 16), lambda i: (0, 0)),
        ],
        out_specs=pl.BlockSpec((_BLK, 16), lambda i: (i, 0)),
        out_shape=jax.ShapeDtypeStruct((n + _BLK, 16), jnp.float32),
    )(x, w1, b1.reshape(1, 32), w2, b2.reshape(1, 16))


def _sumdot(z, w_ref, j):
    return lax.dot_general(_b16(z), _b16(w_ref[j]), (((1,), (1,)), ((), ())),
                           preferred_element_type=jnp.float32)


def _part_specs(parts, nb):
    specs = [pl.BlockSpec(
        (2, _BLK, 16),
        functools.partial(lambda o, m, i: (0, jnp.minimum(i, m - 1) + o, 0),
                          off, nb))
             for _, off in parts]
    return [arr for arr, _ in parts], specs


def _combine(parts, h, wstack, bias):
    n = h.shape[0] - _BLK
    nb = n // _BLK
    arrs, pspecs = _part_specs(parts, nb)

    def body(pa_ref, pb_ref_, pc_ref, h_ref, w_ref, b_ref, o_ref):
        z = b_ref[...]
        for r, p_ref in enumerate((pa_ref, pb_ref_, pc_ref)):
            z = z + _sumdot(p_ref[0] + p_ref[1], w_ref, r)
            z = z + _sumdot(h_ref[...], w_ref, 3 + r)
        z = jnp.maximum(z, 0.0)
        o_ref[...] = jnp.where(pl.program_id(0) < nb, z, 0.0)

    return pl.pallas_call(
        body,
        grid=(nb + 1,),
        in_specs=pspecs + [
            pl.BlockSpec((_BLK, 16), lambda i: (jnp.minimum(i, nb - 1), 0)),
            pl.BlockSpec((6, 16, 16), lambda i: (0, 0, 0)),
            pl.BlockSpec((1, 16), lambda i: (0, 0)),
        ],
        out_specs=pl.BlockSpec((_BLK, 16), lambda i: (i, 0)),
        out_shape=jax.ShapeDtypeStruct((n + _BLK, 16), jnp.float32),
    )(*arrs, h, wstack, bias.reshape(1, 16))


def _final(parts, h, wstack, bias, pw, pb):
    n = h.shape[0] - _BLK
    arrs, pspecs = _part_specs(parts, n // _BLK)

    def body(pa_ref, pb_ref_, pc_ref, h_ref, w_ref, b_ref, pw_ref, pbias_ref,
             o_ref):
        z = b_ref[...]
        for r, p_ref in enumerate((pa_ref, pb_ref_, pc_ref)):
            z = z + _sumdot(p_ref[0] + p_ref[1], w_ref, r)
            z = z + _sumdot(h_ref[...], w_ref, 3 + r)
        h2 = jnp.maximum(z, 0.0)
        prod = _b16(h2).astype(jnp.float32) * _b16(pw_ref[...]).astype(jnp.float32)
        o_ref[...] = jnp.sum(prod, axis=1, keepdims=True) + pbias_ref[0, 0]

    return pl.pallas_call(
        body,
        grid=(n // _BLK,),
        in_specs=pspecs + [
            pl.BlockSpec((_BLK, 16), lambda i: (i, 0)),
            pl.BlockSpec((6, 16, 16), lambda i: (0, 0, 0)),
            pl.BlockSpec((1, 16), lambda i: (0, 0)),
            pl.BlockSpec((1, 16), lambda i: (0, 0)),
            pl.BlockSpec((1, 1), lambda i: (0, 0)),
        ],
        out_specs=pl.BlockSpec((_BLK, 1), lambda i: (i, 0)),
        out_shape=jax.ShapeDtypeStruct((n, 1), jnp.float32),
    )(*arrs, h, wstack, bias.reshape(1, 16), pw, pb.reshape(1, 1))


def _make_segsum(acc_rows, n_sec):
    tot_rows = n_sec * acc_rows
    rpt = tot_rows // 16
    zb = 625
    nz = rpt // zb
    assert nz * zb == rpt
    g32 = 32 * _G

    mesh = plsc.VectorSubcoreMesh(core_axis_name="c", subcore_axis_name="s")

    pb = 7
    nb = _G // pb
    half_rows = pb * 128

    @functools.partial(
        pl.kernel, mesh=mesh,
        compiler_params=pltpu.CompilerParams(use_tc_tiling_on_sc=False),
        out_type=jax.ShapeDtypeStruct((2, tot_rows, 16), jnp.float32),
        scratch_types=[
            pltpu.VMEM((nb, half_rows), jnp.int32),
            pltpu.VMEM((nb, half_rows), jnp.int32),
            pltpu.VMEM((2, half_rows, 16), jnp.float32),
            pltpu.VMEM((zb, 16), jnp.float32),
            pltpu.VMEM_SHARED((tot_rows, 16), jnp.float32),
            pltpu.SemaphoreType.DMA,
            pltpu.SemaphoreType.DMA,
            pltpu.SemaphoreType.DMA,
            pltpu.SemaphoreType.DMA,
        ],
    )
    def k(*refs):
        tables = refs[:n_sec]
        srcg, dstg, out = refs[n_sec], refs[n_sec + 1], refs[n_sec + 2]
        src_v, dst_v, rows_v, buf_v, acc, sg0, sg1, ss0, ss1 = refs[n_sec + 3:]
        cid = lax.axis_index("c")
        sid = lax.axis_index("s")
        wid = cid * 16 + sid

        def zero_row(i, carry):
            buf_v[i] = jnp.zeros((16,), jnp.float32)
            return carry

        lax.fori_loop(0, zb, zero_row, 0)
        for j in range(nz):
            pltpu.sync_copy(buf_v, acc.at[pl.ds(sid * rpt + j * zb, zb)])
        plsc.subcore_barrier()

        for r in range(n_sec):
            tbl = tables[r]
            base_row = (r * 32 + wid) * nb
            pltpu.sync_copy(srcg.at[pl.ds(base_row, nb)], src_v)
            pltpu.sync_copy(dstg.at[pl.ds(base_row, nb)], dst_v)

            sg = (sg0, sg1)
            ss = (ss0, ss1)

            def gather(b, p):
                pltpu.async_copy(tbl.at[src_v.at[b]], rows_v.at[p], sg[p])

            def scatter(b, p):
                pltpu.async_copy(rows_v.at[p], acc.at[dst_v.at[b]],
                                 ss[p], add=True)

            def drain_g(p):
                pltpu.make_async_copy(
                    tbl.at[src_v.at[0]], rows_v.at[p], sg[p]).wait()

            def drain_s(p):
                pltpu.make_async_copy(
                    rows_v.at[p], acc.at[dst_v.at[0]], ss[p]).wait()

            gather(0, 0)
            for b in range(nb):
                p = b % 2
                drain_g(p)
                if b + 1 < nb:
                    if b >= 1:
                        drain_s(1 - p)
                    gather(b + 1, 1 - p)
                scatter(b, p)
            drain_s(nb % 2)
            drain_s(1 - nb % 2)
        plsc.subcore_barrier()

        for j in range(nz):
            row0 = sid * rpt + j * zb
            pltpu.sync_copy(acc.at[pl.ds(row0, zb)], buf_v)
            pltpu.sync_copy(buf_v, out.at[cid, pl.ds(row0, zb)])

    return k


@functools.cache
def _segsum_kernel(acc_rows, n_sec):
    return _make_segsum(acc_rows, n_sec)


def _prep_indices(eis, d, srcs_in_call):
    acc = _ACC[d]
    srcs, dsts = [], []
    for sec, si in enumerate(srcs_in_call):
        s = _NT[si]
        ei = eis[(s, d)].astype(jnp.int32)
        srcs.append(jnp.concatenate(
            [ei[0], jnp.full((_PE - _E,), _N[s], jnp.int32)]))
        dsts.append(jnp.concatenate(
            [ei[1] + sec * acc,
             jnp.full((_PE - _E,), sec * acc, jnp.int32)]))
    return (jnp.concatenate(srcs).reshape(-1, 896),
            jnp.concatenate(dsts).reshape(-1, 896))


def _layer_partials(h, idx, dsts):
    out = {}
    for d in dsts:
        acc = _ACC[d]
        secs = []
        for srcs_in_call, (srcg, dstg) in zip(_SEC[d], idx[d]):
            tables = [h[_NT[si]] for si in srcs_in_call]
            p = _segsum_kernel(acc, len(srcs_in_call))(*tables, srcg, dstg)
            for sec in range(len(srcs_in_call)):
                secs.append((p, sec * acc // _BLK))
        out[d] = secs
    return out


def _wstack(l, d, rel_W, root_W):
    di = _NT.index(d)
    rel_idx = [3 * si + di for si in range(3)]
    return jnp.stack([rel_W[l, r] for r in rel_idx]
                     + [root_W[l, r] for r in rel_idx])


def kernel(x_H, x_C, x_Others, ei_H_H, ei_H_C, ei_H_Others, ei_C_H, ei_C_C,
           ei_C_Others, ei_Others_H, ei_Others_C, ei_Others_Others,
           enc1_W_H, enc1_b_H, enc2_W_H, enc2_b_H,
           enc1_W_C, enc1_b_C, enc2_W_C, enc2_b_C,
           enc1_W_Others, enc1_b_Others, enc2_W_Others, enc2_b_Others,
           rel_W, rel_b, root_W, pred_W_H, pred_b_H, pred_W_C, pred_b_C):
    inp = dict(locals())
    xs = {t: inp[f"x_{t}"] for t in _NT}
    eis = {(s, d): inp[f"ei_{s}_{d}"] for s in _NT for d in _NT}

    h = {t: _encoder(xs[t], inp[f"enc1_W_{t}"], inp[f"enc1_b_{t}"],
                     inp[f"enc2_W_{t}"], inp[f"enc2_b_{t}"]) for t in _NT}

    idx = {d: [_prep_indices(eis, d, call) for call in _SEC[d]] for d in _NT}

    def bias(l, d):
        return jnp.sum(rel_b[l, _NT.index(d)::3], axis=0)

    p1 = _layer_partials(h, idx, _NT)
    h1 = {d: _combine(p1[d], h[d], _wstack(0, d, rel_W, root_W), bias(0, d))
          for d in _NT}

    p2 = _layer_partials(h1, idx, ("H", "C"))
    out_H = _final(p2["H"], h1["H"], _wstack(1, "H", rel_W, root_W),
                   bias(1, "H"), pred_W_H, pred_b_H)
    out_C = _final(p2["C"], h1["C"], _wstack(1, "C", rel_W, root_W),
                   bias(1, "C"), pred_W_C, pred_b_C)
    return out_H, out_C
